# baseline (speedup 1.0000x reference)
import functools
import jax
import jax.numpy as jnp
from jax import lax
from jax.experimental import pallas as pl
from jax.experimental.pallas import tpu as pltpu, tpu_sc as plsc

DIM = 128
VOCAB = 10000
NC, NS = 2, 16
NW = NC * NS
G = 128
NBUF = 6


def _build(B):
    groups_per_w = B // (NW * G)
    assert B % (NW * G) == 0
    n_super = groups_per_w // NBUF
    n_tail = groups_per_w - n_super * NBUF
    rows_per_w = B // NW
    SS = NBUF * G

    mesh = plsc.VectorSubcoreMesh(core_axis_name="c", subcore_axis_name="s")

    scratch = [pltpu.VMEM((2, SS), jnp.int32),
               pltpu.VMEM((max(n_tail, 1) * G,), jnp.int32)]
    scratch += [pltpu.VMEM((G, DIM), jnp.float32) for _ in range(NBUF)]
    scratch += [pltpu.SemaphoreType.DMA for _ in range(2 + 2 * NBUF)]

    @functools.partial(
        pl.kernel,
        out_type=jax.ShapeDtypeStruct((B, DIM), jnp.float32),
        mesh=mesh,
        scratch_types=scratch,
    )
    def k(table_hbm, idx_hbm, out_hbm, idxb, idxt, *scr):
        rows = scr[:NBUF]
        isem = scr[NBUF]
        tsem = scr[NBUF + 1]
        gsem = scr[NBUF + 2:NBUF + 2 + NBUF]
        ssem = scr[NBUF + 2 + NBUF:]
        sid = lax.axis_index("s")
        wid = sid * NC + lax.axis_index("c")
        ibase = wid * rows_per_w
        pltpu.sync_copy(idx_hbm.at[pl.ds(ibase, SS)], idxb.at[0])
        pltpu.async_copy(idx_hbm.at[pl.ds(ibase + SS, SS)], idxb.at[1], isem)
        if n_tail:
            pltpu.async_copy(
                idx_hbm.at[pl.ds(ibase + n_super * SS, n_tail * G)], idxt,
                tsem)
        base = wid * rows_per_w

        for p in range(NBUF):
            pltpu.async_copy(
                table_hbm.at[idxb.at[0, pl.ds(p * G, G)]], rows[p], gsem[p])

        @pl.loop(0, n_super)
        def _(t):
            j0 = t * NBUF
            for p in range(NBUF):
                pltpu.make_async_copy(
                    table_hbm.at[idxb.at[t % 2, pl.ds(p * G, G)]], rows[p],
                    gsem[p]).wait()
                pltpu.async_copy(
                    rows[p], out_hbm.at[pl.ds(base + (j0 + p) * G, G)],
                    ssem[p])

            @pl.when(t < n_super - 1)
            def _():
                pltpu.make_async_copy(
                    idx_hbm.at[pl.ds(ibase + (t + 1) * SS, SS)],
                    idxb.at[(t + 1) % 2], isem).wait()

                @pl.when(t < n_super - 2)
                def _():
                    pltpu.async_copy(
                        idx_hbm.at[pl.ds(ibase + (t + 2) * SS, SS)],
                        idxb.at[t % 2], isem)

                for p in range(NBUF):
                    pltpu.make_async_copy(
                        rows[p], out_hbm.at[pl.ds(base + (j0 + p) * G, G)],
                        ssem[p]).wait()
                    pltpu.async_copy(
                        table_hbm.at[idxb.at[(t + 1) % 2, pl.ds(p * G, G)]],
                        rows[p], gsem[p])

        j0 = (n_super - 1) * NBUF
        if n_tail:
            pltpu.make_async_copy(
                idx_hbm.at[pl.ds(ibase + n_super * SS, n_tail * G)], idxt,
                tsem).wait()
            for p in range(n_tail):
                pltpu.make_async_copy(
                    rows[p], out_hbm.at[pl.ds(base + (j0 + p) * G, G)],
                    ssem[p]).wait()
                pltpu.async_copy(
                    table_hbm.at[idxt.at[pl.ds(p * G, G)]], rows[p], gsem[p])
            for p in range(n_tail):
                jt = n_super * NBUF + p
                pltpu.make_async_copy(
                    table_hbm.at[idxt.at[pl.ds(p * G, G)]], rows[p],
                    gsem[p]).wait()
                pltpu.async_copy(
                    rows[p], out_hbm.at[pl.ds(base + jt * G, G)], ssem[p])
            for p in range(n_tail):
                jt = n_super * NBUF + p
                pltpu.make_async_copy(
                    rows[p], out_hbm.at[pl.ds(base + jt * G, G)],
                    ssem[p]).wait()
            for p in range(n_tail, NBUF):
                pltpu.make_async_copy(
                    rows[p], out_hbm.at[pl.ds(base + (j0 + p) * G, G)],
                    ssem[p]).wait()
        else:
            for p in range(NBUF):
                pltpu.make_async_copy(
                    rows[p], out_hbm.at[pl.ds(base + (j0 + p) * G, G)],
                    ssem[p]).wait()

    return k


_kernel_fn = None


def kernel(x, embedding):
    global _kernel_fn
    B = x.size
    if _kernel_fn is None:
        _kernel_fn = _build(B)
    idx = x.reshape(B).astype(jnp.int32)
    out = _kernel_fn(embedding, idx)
    return out.reshape(x.shape + (DIM,))


# speedup vs baseline: 1.1687x; 1.1687x over previous
import functools
import jax
import jax.numpy as jnp
from jax import lax
from jax.experimental import pallas as pl
from jax.experimental.pallas import tpu as pltpu, tpu_sc as plsc

DIM = 128
VOCAB = 10000
NC, NS = 2, 16
NW = NC * NS
G = 128
NBUF = 2


def _build(B):
    groups_per_w = B // (NW * G)
    assert B % (NW * G) == 0
    n_super = groups_per_w // NBUF
    n_tail = groups_per_w - n_super * NBUF
    rows_per_w = B // NW
    SS = NBUF * G

    mesh = plsc.VectorSubcoreMesh(core_axis_name="c", subcore_axis_name="s")

    scratch = [pltpu.VMEM_SHARED((VOCAB, DIM), jnp.float32),
               pltpu.VMEM((2, SS), jnp.int32),
               pltpu.VMEM((max(n_tail, 1) * G,), jnp.int32)]
    scratch += [pltpu.VMEM((G, DIM), jnp.float32) for _ in range(NBUF)]
    scratch += [pltpu.SemaphoreType.DMA for _ in range(2 + 2 * NBUF)]

    @functools.partial(
        pl.kernel,
        out_type=jax.ShapeDtypeStruct((B, DIM), jnp.float32),
        mesh=mesh,
        scratch_types=scratch,
    )
    def k(table_hbm, idx_hbm, out_hbm, table_sp, idxb, idxt, *scr):
        rows = scr[:NBUF]
        isem = scr[NBUF]
        tsem = scr[NBUF + 1]
        gsem = scr[NBUF + 2:NBUF + 2 + NBUF]
        ssem = scr[NBUF + 2 + NBUF:]
        sid = lax.axis_index("s")
        wid = sid * NC + lax.axis_index("c")
        pltpu.sync_copy(table_hbm.at[pl.ds(sid * 624, 624)],
                        table_sp.at[pl.ds(sid * 624, 624)])

        @pl.when(sid == 0)
        def _():
            pltpu.sync_copy(table_hbm.at[pl.ds(9984, 16)],
                            table_sp.at[pl.ds(9984, 16)])

        ibase = wid * rows_per_w
        pltpu.sync_copy(idx_hbm.at[pl.ds(ibase, SS)], idxb.at[0])
        pltpu.async_copy(idx_hbm.at[pl.ds(ibase + SS, SS)], idxb.at[1], isem)
        if n_tail:
            pltpu.async_copy(
                idx_hbm.at[pl.ds(ibase + n_super * SS, n_tail * G)], idxt,
                tsem)
        plsc.subcore_barrier()
        base = wid * rows_per_w

        for p in range(NBUF):
            pltpu.async_copy(
                table_sp.at[idxb.at[0, pl.ds(p * G, G)]], rows[p], gsem[p])

        @pl.loop(0, n_super)
        def _(t):
            j0 = t * NBUF
            for p in range(NBUF):
                pltpu.make_async_copy(
                    table_sp.at[idxb.at[t % 2, pl.ds(p * G, G)]], rows[p],
                    gsem[p]).wait()
                pltpu.async_copy(
                    rows[p], out_hbm.at[pl.ds(base + (j0 + p) * G, G)],
                    ssem[p])

            @pl.when(t < n_super - 1)
            def _():
                pltpu.make_async_copy(
                    idx_hbm.at[pl.ds(ibase + (t + 1) * SS, SS)],
                    idxb.at[(t + 1) % 2], isem).wait()

                @pl.when(t < n_super - 2)
                def _():
                    pltpu.async_copy(
                        idx_hbm.at[pl.ds(ibase + (t + 2) * SS, SS)],
                        idxb.at[t % 2], isem)

                for p in range(NBUF):
                    pltpu.make_async_copy(
                        rows[p], out_hbm.at[pl.ds(base + (j0 + p) * G, G)],
                        ssem[p]).wait()
                    pltpu.async_copy(
                        table_sp.at[idxb.at[(t + 1) % 2, pl.ds(p * G, G)]],
                        rows[p], gsem[p])

        j0 = (n_super - 1) * NBUF
        if n_tail:
            pltpu.make_async_copy(
                idx_hbm.at[pl.ds(ibase + n_super * SS, n_tail * G)], idxt,
                tsem).wait()
            for p in range(n_tail):
                pltpu.make_async_copy(
                    rows[p], out_hbm.at[pl.ds(base + (j0 + p) * G, G)],
                    ssem[p]).wait()
                pltpu.async_copy(
                    table_sp.at[idxt.at[pl.ds(p * G, G)]], rows[p], gsem[p])
            for p in range(n_tail):
                jt = n_super * NBUF + p
                pltpu.make_async_copy(
                    table_sp.at[idxt.at[pl.ds(p * G, G)]], rows[p],
                    gsem[p]).wait()
                pltpu.async_copy(
                    rows[p], out_hbm.at[pl.ds(base + jt * G, G)], ssem[p])
            for p in range(n_tail):
                jt = n_super * NBUF + p
                pltpu.make_async_copy(
                    rows[p], out_hbm.at[pl.ds(base + jt * G, G)],
                    ssem[p]).wait()
            for p in range(n_tail, NBUF):
                pltpu.make_async_copy(
                    rows[p], out_hbm.at[pl.ds(base + (j0 + p) * G, G)],
                    ssem[p]).wait()
        else:
            for p in range(NBUF):
                pltpu.make_async_copy(
                    rows[p], out_hbm.at[pl.ds(base + (j0 + p) * G, G)],
                    ssem[p]).wait()

    return k


_kernel_fn = None


def kernel(x, embedding):
    global _kernel_fn
    B = x.size
    if _kernel_fn is None:
        _kernel_fn = _build(B)
    idx = x.reshape(B).astype(jnp.int32)
    out = _kernel_fn(embedding, idx)
    return out.reshape(x.shape + (DIM,))


# speedup vs baseline: 1.7161x; 1.4684x over previous
import functools
import jax
import jax.numpy as jnp
from jax import lax
from jax.experimental import pallas as pl
from jax.experimental.pallas import tpu as pltpu, tpu_sc as plsc

DIM = 128
VOCAB = 10000
NC, NS = 2, 16
NW = NC * NS
G = 80
NBUF = 4


def _build(B):
    groups_per_w = B // (NW * G)
    assert B % (NW * G) == 0
    n_super = groups_per_w // NBUF
    n_tail = groups_per_w - n_super * NBUF
    rows_per_w = B // NW
    SS = NBUF * G

    mesh = plsc.VectorSubcoreMesh(core_axis_name="c", subcore_axis_name="s")

    scratch = [pltpu.VMEM_SHARED((VOCAB, DIM), jnp.float32),
               pltpu.VMEM((2 * SS,), jnp.int32),
               pltpu.VMEM((max(n_tail, 1) * G,), jnp.int32)]
    scratch += [pltpu.VMEM((G, DIM), jnp.float32) for _ in range(NBUF)]
    scratch += [pltpu.SemaphoreType.DMA for _ in range(2 + 2 * NBUF)]

    @functools.partial(
        pl.kernel,
        out_type=jax.ShapeDtypeStruct((B, DIM), jnp.float32),
        mesh=mesh,
        scratch_types=scratch,
    )
    def k(table_hbm, idx_hbm, out_hbm, table_sp, idxb, idxt, *scr):
        rows = scr[:NBUF]
        isem = scr[NBUF]
        tsem = scr[NBUF + 1]
        gsem = scr[NBUF + 2:NBUF + 2 + NBUF]
        ssem = scr[NBUF + 2 + NBUF:]
        sid = lax.axis_index("s")
        wid = sid * NC + lax.axis_index("c")
        pltpu.sync_copy(table_hbm.at[pl.ds(sid * 624, 624)],
                        table_sp.at[pl.ds(sid * 624, 624)])

        @pl.when(sid == 0)
        def _():
            pltpu.sync_copy(table_hbm.at[pl.ds(9984, 16)],
                            table_sp.at[pl.ds(9984, 16)])

        ibase = wid * rows_per_w
        pltpu.sync_copy(idx_hbm.at[pl.ds(ibase, SS)], idxb.at[pl.ds(0, SS)])
        pltpu.async_copy(idx_hbm.at[pl.ds(ibase + SS, SS)],
                         idxb.at[pl.ds(SS, SS)], isem)
        if n_tail:
            pltpu.async_copy(
                idx_hbm.at[pl.ds(ibase + n_super * SS, n_tail * G)], idxt,
                tsem)
        plsc.subcore_barrier()
        base = wid * rows_per_w

        for p in range(NBUF):
            pltpu.async_copy(
                table_sp.at[idxb.at[pl.ds(p * G, G)]], rows[p], gsem[p])

        @pl.loop(0, n_super)
        def _(t):
            j0 = t * NBUF
            for p in range(NBUF):
                pltpu.make_async_copy(
                    table_sp.at[idxb.at[pl.ds((t % 2) * SS + p * G, G)]], rows[p],
                    gsem[p]).wait()
                pltpu.async_copy(
                    rows[p], out_hbm.at[pl.ds(base + (j0 + p) * G, G)],
                    ssem[p])

            @pl.when(t < n_super - 1)
            def _():
                pltpu.make_async_copy(
                    idx_hbm.at[pl.ds(ibase + (t + 1) * SS, SS)],
                    idxb.at[pl.ds(((t + 1) % 2) * SS, SS)], isem).wait()

                @pl.when(t < n_super - 2)
                def _():
                    pltpu.async_copy(
                        idx_hbm.at[pl.ds(ibase + (t + 2) * SS, SS)],
                        idxb.at[pl.ds((t % 2) * SS, SS)], isem)

                for p in range(NBUF):
                    pltpu.make_async_copy(
                        rows[p], out_hbm.at[pl.ds(base + (j0 + p) * G, G)],
                        ssem[p]).wait()
                    pltpu.async_copy(
                        table_sp.at[idxb.at[pl.ds(((t + 1) % 2) * SS + p * G, G)]],
                        rows[p], gsem[p])

        j0 = (n_super - 1) * NBUF
        if n_tail:
            pltpu.make_async_copy(
                idx_hbm.at[pl.ds(ibase + n_super * SS, n_tail * G)], idxt,
                tsem).wait()
            for p in range(n_tail):
                pltpu.make_async_copy(
                    rows[p], out_hbm.at[pl.ds(base + (j0 + p) * G, G)],
                    ssem[p]).wait()
                pltpu.async_copy(
                    table_sp.at[idxt.at[pl.ds(p * G, G)]], rows[p], gsem[p])
            for p in range(n_tail):
                jt = n_super * NBUF + p
                pltpu.make_async_copy(
                    table_sp.at[idxt.at[pl.ds(p * G, G)]], rows[p],
                    gsem[p]).wait()
                pltpu.async_copy(
                    rows[p], out_hbm.at[pl.ds(base + jt * G, G)], ssem[p])
            for p in range(n_tail):
                jt = n_super * NBUF + p
                pltpu.make_async_copy(
                    rows[p], out_hbm.at[pl.ds(base + jt * G, G)],
                    ssem[p]).wait()
            for p in range(n_tail, NBUF):
                pltpu.make_async_copy(
                    rows[p], out_hbm.at[pl.ds(base + (j0 + p) * G, G)],
                    ssem[p]).wait()
        else:
            for p in range(NBUF):
                pltpu.make_async_copy(
                    rows[p], out_hbm.at[pl.ds(base + (j0 + p) * G, G)],
                    ssem[p]).wait()

    return k


_kernel_fn = None


def kernel(x, embedding):
    global _kernel_fn
    B = x.size
    if _kernel_fn is None:
        _kernel_fn = _build(B)
    idx = x.reshape(B).astype(jnp.int32)
    out = _kernel_fn(embedding, idx)
    return out.reshape(x.shape + (DIM,))


# speedup vs baseline: 1.7197x; 1.0021x over previous
import functools
import jax
import jax.numpy as jnp
from jax import lax
from jax.experimental import pallas as pl
from jax.experimental.pallas import tpu as pltpu, tpu_sc as plsc

DIM = 128
VOCAB = 10000
NC, NS = 2, 16
NW = NC * NS
G = 64
NBUF = 5


def _build(B):
    groups_per_w = B // (NW * G)
    assert B % (NW * G) == 0
    n_super = groups_per_w // NBUF
    n_tail = groups_per_w - n_super * NBUF
    rows_per_w = B // NW
    SS = NBUF * G

    mesh = plsc.VectorSubcoreMesh(core_axis_name="c", subcore_axis_name="s")

    scratch = [pltpu.VMEM_SHARED((VOCAB, DIM), jnp.float32),
               pltpu.VMEM((2 * SS,), jnp.int32),
               pltpu.VMEM((max(n_tail, 1) * G,), jnp.int32)]
    scratch += [pltpu.VMEM((G, DIM), jnp.float32) for _ in range(NBUF)]
    scratch += [pltpu.SemaphoreType.DMA for _ in range(2 + 2 * NBUF)]

    @functools.partial(
        pl.kernel,
        out_type=jax.ShapeDtypeStruct((B, DIM), jnp.float32),
        mesh=mesh,
        scratch_types=scratch,
    )
    def k(table_hbm, idx_hbm, out_hbm, table_sp, idxb, idxt, *scr):
        rows = scr[:NBUF]
        isem = scr[NBUF]
        tsem = scr[NBUF + 1]
        gsem = scr[NBUF + 2:NBUF + 2 + NBUF]
        ssem = scr[NBUF + 2 + NBUF:]
        sid = lax.axis_index("s")
        wid = sid * NC + lax.axis_index("c")
        pltpu.sync_copy(table_hbm.at[pl.ds(sid * 624, 624)],
                        table_sp.at[pl.ds(sid * 624, 624)])

        @pl.when(sid == 0)
        def _():
            pltpu.sync_copy(table_hbm.at[pl.ds(9984, 16)],
                            table_sp.at[pl.ds(9984, 16)])

        ibase = wid * rows_per_w
        pltpu.sync_copy(idx_hbm.at[pl.ds(ibase, SS)], idxb.at[pl.ds(0, SS)])
        pltpu.async_copy(idx_hbm.at[pl.ds(ibase + SS, SS)],
                         idxb.at[pl.ds(SS, SS)], isem)
        if n_tail:
            pltpu.async_copy(
                idx_hbm.at[pl.ds(ibase + n_super * SS, n_tail * G)], idxt,
                tsem)
        plsc.subcore_barrier()
        base = wid * rows_per_w

        for p in range(NBUF):
            pltpu.async_copy(
                table_sp.at[idxb.at[pl.ds(p * G, G)]], rows[p], gsem[p])

        @pl.loop(0, n_super)
        def _(t):
            j0 = t * NBUF
            for p in range(NBUF):
                pltpu.make_async_copy(
                    table_sp.at[idxb.at[pl.ds((t % 2) * SS + p * G, G)]], rows[p],
                    gsem[p]).wait()
                pltpu.async_copy(
                    rows[p], out_hbm.at[pl.ds(base + (j0 + p) * G, G)],
                    ssem[p])

            @pl.when(t < n_super - 1)
            def _():
                pltpu.make_async_copy(
                    idx_hbm.at[pl.ds(ibase + (t + 1) * SS, SS)],
                    idxb.at[pl.ds(((t + 1) % 2) * SS, SS)], isem).wait()

                @pl.when(t < n_super - 2)
                def _():
                    pltpu.async_copy(
                        idx_hbm.at[pl.ds(ibase + (t + 2) * SS, SS)],
                        idxb.at[pl.ds((t % 2) * SS, SS)], isem)

                for p in range(NBUF):
                    pltpu.make_async_copy(
                        rows[p], out_hbm.at[pl.ds(base + (j0 + p) * G, G)],
                        ssem[p]).wait()
                    pltpu.async_copy(
                        table_sp.at[idxb.at[pl.ds(((t + 1) % 2) * SS + p * G, G)]],
                        rows[p], gsem[p])

        j0 = (n_super - 1) * NBUF
        if n_tail:
            pltpu.make_async_copy(
                idx_hbm.at[pl.ds(ibase + n_super * SS, n_tail * G)], idxt,
                tsem).wait()
            for p in range(n_tail):
                pltpu.make_async_copy(
                    rows[p], out_hbm.at[pl.ds(base + (j0 + p) * G, G)],
                    ssem[p]).wait()
                pltpu.async_copy(
                    table_sp.at[idxt.at[pl.ds(p * G, G)]], rows[p], gsem[p])
            for p in range(n_tail):
                jt = n_super * NBUF + p
                pltpu.make_async_copy(
                    table_sp.at[idxt.at[pl.ds(p * G, G)]], rows[p],
                    gsem[p]).wait()
                pltpu.async_copy(
                    rows[p], out_hbm.at[pl.ds(base + jt * G, G)], ssem[p])
            for p in range(n_tail):
                jt = n_super * NBUF + p
                pltpu.make_async_copy(
                    rows[p], out_hbm.at[pl.ds(base + jt * G, G)],
                    ssem[p]).wait()
            for p in range(n_tail, NBUF):
                pltpu.make_async_copy(
                    rows[p], out_hbm.at[pl.ds(base + (j0 + p) * G, G)],
                    ssem[p]).wait()
        else:
            for p in range(NBUF):
                pltpu.make_async_copy(
                    rows[p], out_hbm.at[pl.ds(base + (j0 + p) * G, G)],
                    ssem[p]).wait()

    return k


_kernel_fn = None


def kernel(x, embedding):
    global _kernel_fn
    B = x.size
    if _kernel_fn is None:
        _kernel_fn = _build(B)
    idx = x.reshape(B).astype(jnp.int32)
    out = _kernel_fn(embedding, idx)
    return out.reshape(x.shape + (DIM,))


# speedup vs baseline: 1.7272x; 1.0044x over previous
import functools
import jax
import jax.numpy as jnp
from jax import lax
from jax.experimental import pallas as pl
from jax.experimental.pallas import tpu as pltpu, tpu_sc as plsc

DIM = 128
VOCAB = 10000
NC, NS = 2, 16
NW = NC * NS
G = 64
NBUF = 6


def _build(B):
    groups_per_w = B // (NW * G)
    assert B % (NW * G) == 0
    n_super = groups_per_w // NBUF
    n_tail = groups_per_w - n_super * NBUF
    rows_per_w = B // NW
    SS = NBUF * G

    mesh = plsc.VectorSubcoreMesh(core_axis_name="c", subcore_axis_name="s")

    scratch = [pltpu.VMEM_SHARED((VOCAB, DIM), jnp.float32),
               pltpu.VMEM((2 * SS,), jnp.int32),
               pltpu.VMEM((max(n_tail, 1) * G,), jnp.int32)]
    scratch += [pltpu.VMEM((G, DIM), jnp.float32) for _ in range(NBUF)]
    scratch += [pltpu.SemaphoreType.DMA for _ in range(2 + 2 * NBUF)]

    @functools.partial(
        pl.kernel,
        out_type=jax.ShapeDtypeStruct((B, DIM), jnp.float32),
        mesh=mesh,
        scratch_types=scratch,
    )
    def k(table_hbm, idx_hbm, out_hbm, table_sp, idxb, idxt, *scr):
        rows = scr[:NBUF]
        isem = scr[NBUF]
        tsem = scr[NBUF + 1]
        gsem = scr[NBUF + 2:NBUF + 2 + NBUF]
        ssem = scr[NBUF + 2 + NBUF:]
        sid = lax.axis_index("s")
        wid = sid * NC + lax.axis_index("c")
        pltpu.sync_copy(table_hbm.at[pl.ds(sid * 624, 624)],
                        table_sp.at[pl.ds(sid * 624, 624)])

        @pl.when(sid == 0)
        def _():
            pltpu.sync_copy(table_hbm.at[pl.ds(9984, 16)],
                            table_sp.at[pl.ds(9984, 16)])

        ibase = wid * rows_per_w
        pltpu.sync_copy(idx_hbm.at[pl.ds(ibase, SS)], idxb.at[pl.ds(0, SS)])
        pltpu.async_copy(idx_hbm.at[pl.ds(ibase + SS, SS)],
                         idxb.at[pl.ds(SS, SS)], isem)
        if n_tail:
            pltpu.async_copy(
                idx_hbm.at[pl.ds(ibase + n_super * SS, n_tail * G)], idxt,
                tsem)
        plsc.subcore_barrier()
        base = wid * rows_per_w

        for p in range(NBUF):
            pltpu.async_copy(
                table_sp.at[idxb.at[pl.ds(p * G, G)]], rows[p], gsem[p])

        @pl.loop(0, n_super)
        def _(t):
            j0 = t * NBUF
            for p in range(NBUF):
                pltpu.make_async_copy(
                    table_sp.at[idxb.at[pl.ds((t % 2) * SS + p * G, G)]], rows[p],
                    gsem[p]).wait()
                pltpu.async_copy(
                    rows[p], out_hbm.at[pl.ds(base + (j0 + p) * G, G)],
                    ssem[p])

            @pl.when(t < n_super - 1)
            def _():
                pltpu.make_async_copy(
                    idx_hbm.at[pl.ds(ibase + (t + 1) * SS, SS)],
                    idxb.at[pl.ds(((t + 1) % 2) * SS, SS)], isem).wait()

                @pl.when(t < n_super - 2)
                def _():
                    pltpu.async_copy(
                        idx_hbm.at[pl.ds(ibase + (t + 2) * SS, SS)],
                        idxb.at[pl.ds((t % 2) * SS, SS)], isem)

                for p in range(NBUF):
                    pltpu.make_async_copy(
                        rows[p], out_hbm.at[pl.ds(base + (j0 + p) * G, G)],
                        ssem[p]).wait()
                    pltpu.async_copy(
                        table_sp.at[idxb.at[pl.ds(((t + 1) % 2) * SS + p * G, G)]],
                        rows[p], gsem[p])

        j0 = (n_super - 1) * NBUF
        if n_tail:
            pltpu.make_async_copy(
                idx_hbm.at[pl.ds(ibase + n_super * SS, n_tail * G)], idxt,
                tsem).wait()
            for p in range(n_tail):
                pltpu.make_async_copy(
                    rows[p], out_hbm.at[pl.ds(base + (j0 + p) * G, G)],
                    ssem[p]).wait()
                pltpu.async_copy(
                    table_sp.at[idxt.at[pl.ds(p * G, G)]], rows[p], gsem[p])
            for p in range(n_tail):
                jt = n_super * NBUF + p
                pltpu.make_async_copy(
                    table_sp.at[idxt.at[pl.ds(p * G, G)]], rows[p],
                    gsem[p]).wait()
                pltpu.async_copy(
                    rows[p], out_hbm.at[pl.ds(base + jt * G, G)], ssem[p])
            for p in range(n_tail):
                jt = n_super * NBUF + p
                pltpu.make_async_copy(
                    rows[p], out_hbm.at[pl.ds(base + jt * G, G)],
                    ssem[p]).wait()
            for p in range(n_tail, NBUF):
                pltpu.make_async_copy(
                    rows[p], out_hbm.at[pl.ds(base + (j0 + p) * G, G)],
                    ssem[p]).wait()
        else:
            for p in range(NBUF):
                pltpu.make_async_copy(
                    rows[p], out_hbm.at[pl.ds(base + (j0 + p) * G, G)],
                    ssem[p]).wait()

    return k


_kernel_fn = None


def kernel(x, embedding):
    global _kernel_fn
    B = x.size
    if _kernel_fn is None:
        _kernel_fn = _build(B)
    idx = x.reshape(B).astype(jnp.int32)
    out = _kernel_fn(embedding, idx)
    return out.reshape(x.shape + (DIM,))
